# Initial kernel scaffold; baseline (speedup 1.0000x reference)
import jax
import jax.numpy as jnp
from jax.experimental import pallas as pl


def kernel(x, level_weight):
    raise NotImplementedError("write your pallas kernel here")


# speedup vs baseline: 8.6931x; 8.6931x over previous
import jax
import jax.numpy as jnp
from jax import lax
from jax.experimental import pallas as pl
from jax.experimental.pallas import tpu as pltpu

_LEVELS = 1024
_OUT = 2048
_N = 224 * 224
_CHUNK = 1024
_NSTEPS = _N // _CHUNK


def _body(x_ref, w_ref, o_ref, counts_ref):
    i = pl.program_id(0)

    @pl.when(i == 0)
    def _init():
        counts_ref[...] = jnp.zeros_like(counts_ref)

    xb = x_ref[...]
    idx = jnp.clip(jnp.round(xb * (_LEVELS - 1)).astype(jnp.int32), 0, _LEVELS - 1)
    levels = lax.broadcasted_iota(jnp.int32, (1, _LEVELS), 1)
    eq = (idx == levels).astype(jnp.float32)
    counts_ref[...] += jnp.sum(eq, axis=0, keepdims=True)

    @pl.when(i == _NSTEPS - 1)
    def _fin():
        o_ref[...] = jnp.dot(counts_ref[...], w_ref[...],
                             preferred_element_type=jnp.float32)


def kernel(x, level_weight):
    x2 = x.reshape(_N, 1)
    out = pl.pallas_call(
        _body,
        grid=(_NSTEPS,),
        in_specs=[
            pl.BlockSpec((_CHUNK, 1), lambda i: (i, 0)),
            pl.BlockSpec((_LEVELS, _OUT), lambda i: (0, 0)),
        ],
        out_specs=pl.BlockSpec((1, _OUT), lambda i: (0, 0)),
        out_shape=jax.ShapeDtypeStruct((1, _OUT), jnp.float32),
        scratch_shapes=[pltpu.VMEM((1, _LEVELS), jnp.float32)],
    )(x2, level_weight)
    return out.reshape(_OUT)


# speedup vs baseline: 17.5875x; 2.0232x over previous
import functools

import jax
import jax.numpy as jnp
from jax import lax
from jax.experimental import pallas as pl
from jax.experimental.pallas import tpu as pltpu
from jax.experimental.pallas import tpu_sc as plsc

_LEVELS = 1024
_OUT = 2048
_N = 224 * 224
_NW = 32
_PER_W = _N // _NW
_VPW = _PER_W // 16
_NLANES = 16
_RNE = 8388608.0

_mesh = plsc.VectorSubcoreMesh(core_axis_name="c", subcore_axis_name="s")


@functools.partial(
    pl.kernel,
    mesh=_mesh,
    out_type=jax.ShapeDtypeStruct((_NW, _LEVELS), jnp.float32),
    compiler_params=pltpu.CompilerParams(needs_layout_passes=False),
    scratch_types=[
        pltpu.VMEM((_PER_W,), jnp.float32),
        pltpu.VMEM((_NLANES * _LEVELS,), jnp.float32),
        pltpu.VMEM((_LEVELS,), jnp.float32),
    ],
)
def _sc_hist(x_hbm, out_hbm, x_v, tab_v, cnt_v):
    wid = lax.axis_index("s") * 2 + lax.axis_index("c")
    base = wid * _PER_W
    pltpu.sync_copy(x_hbm.at[pl.ds(base, _PER_W)], x_v)

    zeros16 = jnp.zeros((16,), jnp.float32)

    def _zero(j, c):
        for t in range(_NLANES):
            tab_v[pl.ds(t * _LEVELS + j * 16, 16)] = zeros16
        return c

    lax.fori_loop(0, _LEVELS // 16, _zero, 0)

    lane_base = lax.iota(jnp.int32, 16) * _LEVELS
    ones16 = jnp.ones((16,), jnp.float32)

    def _hist(i, c):
        xv = x_v[pl.ds(i * 16, 16)]
        v = xv * float(_LEVELS - 1)
        r = (v + _RNE) - _RNE
        idx = jnp.clip(r.astype(jnp.int32), 0, _LEVELS - 1)
        plsc.addupdate_scatter(tab_v, [lane_base + idx], ones16)
        return c

    lax.fori_loop(0, _VPW, _hist, 0)

    def _red(j, c):
        acc = tab_v[pl.ds(j * 16, 16)]
        for t in range(1, _NLANES):
            acc = acc + tab_v[pl.ds(t * _LEVELS + j * 16, 16)]
        cnt_v[pl.ds(j * 16, 16)] = acc
        return c

    lax.fori_loop(0, _LEVELS // 16, _red, 0)

    pltpu.sync_copy(cnt_v, out_hbm.at[wid])


def _mv_body(cp_ref, w_ref, o_ref):
    c = jnp.sum(cp_ref[...], axis=0, keepdims=True)
    o_ref[...] = jnp.dot(c, w_ref[...], preferred_element_type=jnp.float32)


def kernel(x, level_weight):
    counts_parts = _sc_hist(x)
    out = pl.pallas_call(
        _mv_body,
        out_shape=jax.ShapeDtypeStruct((1, _OUT), jnp.float32),
    )(counts_parts, level_weight)
    return out.reshape(_OUT)
